# baseline (speedup 1.0000x reference)
import jax
import jax.numpy as jnp
from jax import lax
from jax.experimental import pallas as pl
from jax.experimental.pallas import tpu as pltpu
from jax.experimental.pallas import tpu_sc as plsc

_NC = 2
_NS = 16
_NW = _NC * _NS
_B = 16384
_CH = 128
_NROWS = _B // _CH
_RPW = _NROWS // _NW
_DIM = 10
_PAD = 16
_N_TBL = 100000
_PBLK = 10000


def _sc_gather_body(u_hbm, v_hbm, users_hbm, items_hbm,
                    pu_out, pv_out,
                    idx_u, idx_v, rows_u, rows_v, sem):
    wid = lax.axis_index("s") * _NC + lax.axis_index("c")
    base = wid * _RPW
    pltpu.sync_copy(u_hbm.at[pl.ds(base, _RPW)], idx_u)
    pltpu.sync_copy(v_hbm.at[pl.ds(base, _RPW)], idx_v)
    copies = []
    for j in range(_RPW):
        copies.append(pltpu.async_copy(users_hbm.at[idx_u.at[j]], rows_u.at[j], sem))
        copies.append(pltpu.async_copy(items_hbm.at[idx_v.at[j]], rows_v.at[j], sem))
    for cp in copies:
        cp.wait()
    for j in range(_RPW):
        pltpu.sync_copy(rows_u.at[j], pu_out.at[pl.ds((base + j) * _CH, _CH)])
        pltpu.sync_copy(rows_v.at[j], pv_out.at[pl.ds((base + j) * _CH, _CH)])


_sc_gather = pl.kernel(
    _sc_gather_body,
    out_type=(
        jax.ShapeDtypeStruct((_B, _PAD), jnp.float32),
        jax.ShapeDtypeStruct((_B, _PAD), jnp.float32),
    ),
    mesh=plsc.VectorSubcoreMesh(core_axis_name="c", subcore_axis_name="s",
                                num_cores=_NC, num_subcores=_NS),
    scratch_types=[
        pltpu.VMEM((_RPW, _CH), jnp.int32),
        pltpu.VMEM((_RPW, _CH), jnp.int32),
        pltpu.VMEM((_RPW, _CH, _PAD), jnp.float32),
        pltpu.VMEM((_RPW, _CH, _PAD), jnp.float32),
        pltpu.SemaphoreType.DMA,
    ],
    compiler_params=pltpu.CompilerParams(use_tc_tiling_on_sc=False),
)


_BLK = 4096


def _tc_dense_body(pu_ref, pv_ref,
                   w1_ref, b1_ref, w2_ref, b2_ref, w4_ref, b4_ref, out_ref):
    pu = pu_ref[:, :_DIM]
    pv = pv_ref[:, :_DIM]
    bu = pu_ref[:, _DIM:_DIM + 1]
    bv = pv_ref[:, _DIM:_DIM + 1]
    nu = jnp.sqrt(jnp.sum(pu * pu, axis=1, keepdims=True))
    nv = jnp.sqrt(jnp.sum(pv * pv, axis=1, keepdims=True))
    su = jnp.minimum(1.0, 1.0 / jnp.maximum(nu, 1e-7))
    sv = jnp.minimum(1.0, 1.0 / jnp.maximum(nv, 1e-7))
    x = (pu * su) * (pv * sv)
    h = jnp.dot(x, w1_ref[...], preferred_element_type=jnp.float32) + b1_ref[...]
    h = jnp.dot(h, w2_ref[...], preferred_element_type=jnp.float32) + b2_ref[...]
    h = jax.nn.sigmoid(h)
    o = jnp.dot(h, w4_ref[...], preferred_element_type=jnp.float32)
    out_ref[...] = o + b4_ref[0, 0] + bu + bv


def _tc_dense(pu, pv, W1, b1, W2, b2, W4, b4):
    grid = (_B // _BLK,)
    return pl.pallas_call(
        _tc_dense_body,
        grid=grid,
        in_specs=[
            pl.BlockSpec((_BLK, _PAD), lambda i: (i, 0)),
            pl.BlockSpec((_BLK, _PAD), lambda i: (i, 0)),
            pl.BlockSpec((_DIM, 128), lambda i: (0, 0)),
            pl.BlockSpec((128,), lambda i: (0,)),
            pl.BlockSpec((128, 64), lambda i: (0, 0)),
            pl.BlockSpec((64,), lambda i: (0,)),
            pl.BlockSpec((64, 1), lambda i: (0, 0)),
            pl.BlockSpec((1, 1), lambda i: (0, 0)),
        ],
        out_specs=pl.BlockSpec((_BLK, 1), lambda i: (i, 0)),
        out_shape=jax.ShapeDtypeStruct((_B, 1), jnp.float32),
    )(pu, pv, W1, b1, W2, b2, W4, b4)


def kernel(u, v, users, items, u_bias, i_bias, W1, b1, W2, b2, W4, b4):
    n_u = users.shape[0]
    n_i = items.shape[0]
    zu = jnp.zeros((n_u, _PAD - _DIM - 1), jnp.float32)
    zi = jnp.zeros((n_i, _PAD - _DIM - 1), jnp.float32)
    users16 = jnp.concatenate([users, u_bias, zu], axis=1)
    items16 = jnp.concatenate([items, i_bias, zi], axis=1)
    u2 = u.astype(jnp.int32).reshape(_NROWS, _CH)
    v2 = v.astype(jnp.int32).reshape(_NROWS, _CH)
    pu, pv = _sc_gather(u2, v2, users16, items16)
    out = _tc_dense(pu, pv, W1, b1, W2, b2, W4, b4.reshape(1, 1))
    return out.reshape(_B)


# speedup vs baseline: 1.0022x; 1.0022x over previous
import jax
import jax.numpy as jnp
from jax import lax
from jax.experimental import pallas as pl
from jax.experimental.pallas import tpu as pltpu
from jax.experimental.pallas import tpu_sc as plsc

_NC = 2
_NS = 16
_NW = _NC * _NS
_B = 16384
_CH = 128
_NROWS = _B // _CH
_RPW = _NROWS // _NW
_DIM = 10
_PAD = 16
_N_TBL = 100000
_PBLK = 10000


def _sc_gather_body(u_hbm, v_hbm, users_hbm, items_hbm,
                    pu_out, pv_out,
                    idx_u, idx_v, rows_u, rows_v, sem):
    wid = lax.axis_index("s") * _NC + lax.axis_index("c")
    base = wid * _RPW
    pltpu.sync_copy(u_hbm.at[pl.ds(base, _RPW)], idx_u)
    pltpu.sync_copy(v_hbm.at[pl.ds(base, _RPW)], idx_v)
    copies = []
    for j in range(_RPW):
        copies.append(pltpu.async_copy(users_hbm.at[idx_u.at[j]], rows_u.at[j], sem))
        copies.append(pltpu.async_copy(items_hbm.at[idx_v.at[j]], rows_v.at[j], sem))
    for cp in copies:
        cp.wait()
    for j in range(_RPW):
        pltpu.sync_copy(rows_u.at[j], pu_out.at[pl.ds((base + j) * _CH, _CH)])
        pltpu.sync_copy(rows_v.at[j], pv_out.at[pl.ds((base + j) * _CH, _CH)])


def _sc_gather(u2, v2, users16, items16):
    gather = pl.kernel(
        _sc_gather_body,
        out_type=(
            jax.ShapeDtypeStruct((_B, _PAD), jnp.float32),
            jax.ShapeDtypeStruct((_B, _PAD), jnp.float32),
        ),
        mesh=plsc.VectorSubcoreMesh(core_axis_name="c", subcore_axis_name="s",
                                    num_cores=_NC, num_subcores=_NS),
        scratch_types=[
            pltpu.VMEM((_RPW, _CH), jnp.int32),
            pltpu.VMEM((_RPW, _CH), jnp.int32),
            pltpu.VMEM((_RPW, _CH, _PAD), jnp.float32),
            pltpu.VMEM((_RPW, _CH, _PAD), jnp.float32),
            pltpu.SemaphoreType.DMA,
        ],
        compiler_params=pltpu.CompilerParams(use_tc_tiling_on_sc=False),
    )
    return gather(u2, v2, users16, items16)


_BLK = 4096


def _tc_dense_body(pu_ref, pv_ref,
                   w1_ref, b1_ref, w2_ref, b2_ref, w4_ref, b4_ref, out_ref):
    pu = pu_ref[:, :_DIM]
    pv = pv_ref[:, :_DIM]
    bu = pu_ref[:, _DIM:_DIM + 1]
    bv = pv_ref[:, _DIM:_DIM + 1]
    nu = jnp.sqrt(jnp.sum(pu * pu, axis=1, keepdims=True))
    nv = jnp.sqrt(jnp.sum(pv * pv, axis=1, keepdims=True))
    su = jnp.minimum(1.0, 1.0 / jnp.maximum(nu, 1e-7))
    sv = jnp.minimum(1.0, 1.0 / jnp.maximum(nv, 1e-7))
    x = (pu * su) * (pv * sv)
    h = jnp.dot(x, w1_ref[...], preferred_element_type=jnp.float32) + b1_ref[...]
    h = jnp.dot(h, w2_ref[...], preferred_element_type=jnp.float32) + b2_ref[...]
    h = jax.nn.sigmoid(h)
    o = jnp.dot(h, w4_ref[...], preferred_element_type=jnp.float32)
    out_ref[...] = o + b4_ref[0, 0] + bu + bv


def _tc_dense(pu, pv, W1, b1, W2, b2, W4, b4):
    grid = (_B // _BLK,)
    return pl.pallas_call(
        _tc_dense_body,
        grid=grid,
        in_specs=[
            pl.BlockSpec((_BLK, _PAD), lambda i: (i, 0)),
            pl.BlockSpec((_BLK, _PAD), lambda i: (i, 0)),
            pl.BlockSpec((_DIM, 128), lambda i: (0, 0)),
            pl.BlockSpec((128,), lambda i: (0,)),
            pl.BlockSpec((128, 64), lambda i: (0, 0)),
            pl.BlockSpec((64,), lambda i: (0,)),
            pl.BlockSpec((64, 1), lambda i: (0, 0)),
            pl.BlockSpec((1, 1), lambda i: (0, 0)),
        ],
        out_specs=pl.BlockSpec((_BLK, 1), lambda i: (i, 0)),
        out_shape=jax.ShapeDtypeStruct((_B, 1), jnp.float32),
    )(pu, pv, W1, b1, W2, b2, W4, b4)


def kernel(u, v, users, items, u_bias, i_bias, W1, b1, W2, b2, W4, b4):
    n_u = users.shape[0]
    n_i = items.shape[0]
    zu = jnp.zeros((n_u, _PAD - _DIM - 1), jnp.float32)
    zi = jnp.zeros((n_i, _PAD - _DIM - 1), jnp.float32)
    users16 = jnp.concatenate([users, u_bias, zu], axis=1)
    items16 = jnp.concatenate([items, i_bias, zi], axis=1)
    u2 = u.astype(jnp.int32).reshape(_NROWS, _CH)
    v2 = v.astype(jnp.int32).reshape(_NROWS, _CH)
    pu, pv = _sc_gather(u2, v2, users16, items16)
    out = _tc_dense(pu, pv, W1, b1, W2, b2, W4, b4.reshape(1, 1))
    return out.reshape(_B)
